# Initial kernel scaffold; baseline (speedup 1.0000x reference)
import jax
import jax.numpy as jnp
from jax.experimental import pallas as pl


def kernel(x, edge_index, edge_attr, W_e1, b_e1, W_l1, b_l1, W_r1, b_r1, W_e2, b_e2, W_l2, b_l2, W_r2, b_r2):
    raise NotImplementedError("write your pallas kernel here")


# speedup vs baseline: 6.2478x; 6.2478x over previous
import functools

import jax
import jax.numpy as jnp
from jax import lax
from jax.experimental import pallas as pl
from jax.experimental.pallas import tpu as pltpu
from jax.experimental.pallas import tpu_sc as plsc

_NC = 2
_NS = 16
_NW = _NC * _NS
_ROW = 128
_K = 8


def _seg_sum_sc(n_acc, e_rows, width, with_ea):
  rows_per_w = e_rows // _NW
  n_blocks = rows_per_w // _K
  zrows = n_acc // _NS

  mesh = plsc.VectorSubcoreMesh(
      core_axis_name="c", subcore_axis_name="s",
      num_cores=_NC, num_subcores=_NS)

  out_type = [jax.ShapeDtypeStruct((_NC, n_acc, width), jnp.float32)]
  scratch = [
      pltpu.VMEM((_K, _ROW), jnp.int32),
      pltpu.VMEM((_K, _ROW), jnp.int32),
      pltpu.VMEM((_K, _ROW, width), jnp.float32),
      pltpu.VMEM_SHARED((n_acc, width), jnp.float32),
      pltpu.SemaphoreType.DMA,
  ]
  if with_ea:
    out_type.append(jax.ShapeDtypeStruct((_NC, n_acc, 16), jnp.float32))
    scratch.append(pltpu.VMEM((_K, _ROW, 16), jnp.float32))
    scratch.append(pltpu.VMEM_SHARED((n_acc, 16), jnp.float32))

  def body(*refs):
    if with_ea:
      (table, srcr, dstr, ear, z_s, s_out, a_out,
       src_v, dst_v, rows_v, acc_s, sem, ea_v, acc_a) = refs
    else:
      (table, srcr, dstr, z_s, s_out,
       src_v, dst_v, rows_v, acc_s, sem) = refs

    cid = lax.axis_index("c")
    sid = lax.axis_index("s")
    wid = sid * _NC + cid

    z0 = sid * zrows
    pltpu.sync_copy(z_s.at[pl.ds(z0, zrows)], acc_s.at[pl.ds(z0, zrows)])
    if with_ea:
      pltpu.sync_copy(z_s.at[pl.ds(z0, zrows)], acc_a.at[pl.ds(z0, zrows)])
    plsc.subcore_barrier()

    row0 = wid * rows_per_w

    @pl.loop(0, n_blocks)
    def _block(b):
      base = row0 + b * _K
      pltpu.sync_copy(srcr.at[pl.ds(base, _K)], src_v)
      pltpu.sync_copy(dstr.at[pl.ds(base, _K)], dst_v)
      if with_ea:
        pltpu.sync_copy(ear.at[pl.ds(base, _K)], ea_v)
      for j in range(_K):
        pltpu.async_copy(table.at[src_v.at[j]], rows_v.at[j], sem).wait()
        pltpu.sync_copy(rows_v.at[j], acc_s.at[dst_v.at[j]], add=True)
        if with_ea:
          pltpu.sync_copy(ea_v.at[j], acc_a.at[dst_v.at[j]], add=True)

    plsc.subcore_barrier()

    pltpu.sync_copy(acc_s.at[pl.ds(z0, zrows)],
                    s_out.at[cid, pl.ds(z0, zrows)])
    if with_ea:
      pltpu.sync_copy(acc_a.at[pl.ds(z0, zrows)],
                      a_out.at[cid, pl.ds(z0, zrows)])

  return pl.kernel(
      body, out_type=out_type, mesh=mesh, scratch_types=scratch,
      compiler_params=pltpu.CompilerParams(use_tc_tiling_on_sc=False))


def _pre_tc(x_ref, wl_ref, wr_ref, we_ref, be_ref,
            xl_ref, xr_ref, m1_ref, c1_ref):
  xv = x_ref[...]
  wl = wl_ref[...]
  xl_ref[...] = jnp.dot(xv, wl, preferred_element_type=jnp.float32)
  xr_ref[...] = jnp.dot(xv, wr_ref[...], preferred_element_type=jnp.float32)
  m1_ref[...] = jnp.dot(we_ref[...], wl, preferred_element_type=jnp.float32)
  c1_ref[...] = jnp.dot(be_ref[...], wl, preferred_element_type=jnp.float32)


def _mid_tc(s1a_ref, s1b_ref, a4a_ref, a4b_ref, xr_ref, m1_ref, c1_ref,
            b1_ref, wr2_ref, h_ref, hr_ref):
  nn = h_ref.shape[0]
  a4 = a4a_ref[:nn, :] + a4b_ref[:nn, :]
  cnt = a4[:, 3:4]
  m1 = m1_ref[...]
  ce = (a4[:, 0:1] * m1[0:1, :] + a4[:, 1:2] * m1[1:2, :]
        + a4[:, 2:3] * m1[2:3, :])
  num = s1a_ref[:nn, :] + s1b_ref[:nn, :] + ce + cnt * c1_ref[...]
  h = jnp.maximum(num / jnp.maximum(cnt, 1.0) + xr_ref[...] + b1_ref[...], 0.0)
  h_ref[...] = h
  hr_ref[...] = jnp.dot(h, wr2_ref[...], preferred_element_type=jnp.float32)


def _out_tc(s2a_ref, s2b_ref, a4a_ref, a4b_ref, hr_ref, we2_ref, be2_ref,
            wl2_ref, b2_ref, o_ref):
  nn = o_ref.shape[0]
  a4 = a4a_ref[:nn, :] + a4b_ref[:nn, :]
  cnt = a4[:, 3:4]
  we2 = we2_ref[...]
  ce = (a4[:, 0:1] * we2[0:1, :] + a4[:, 1:2] * we2[1:2, :]
        + a4[:, 2:3] * we2[2:3, :])
  t = s2a_ref[:nn, :] + s2b_ref[:nn, :] + ce + cnt * be2_ref[...]
  agg = t / jnp.maximum(cnt, 1.0)
  o_ref[...] = (jnp.dot(agg, wl2_ref[...], preferred_element_type=jnp.float32)
                + hr_ref[...] + b2_ref[...])


def kernel(x, edge_index, edge_attr, W_e1, b_e1, W_l1, b_l1, W_r1, b_r1,
           W_e2, b_e2, W_l2, b_l2, W_r2, b_r2):
  n, d_in = x.shape
  e = edge_index.shape[1]
  h_dim = W_l1.shape[1]
  d_out = W_l2.shape[1]

  blk = _NW * _K * _ROW
  e_pad = -(-e // blk) * blk
  pad = e_pad - e
  e_rows = e_pad // _ROW
  n_acc = -(-(n + 1) // (_NS * 8)) * (_NS * 8)

  src = edge_index[0]
  dst = edge_index[1]
  if pad:
    src = jnp.concatenate([src, jnp.zeros((pad,), jnp.int32)])
    dst = jnp.concatenate([dst, jnp.full((pad,), n, jnp.int32)])
  src2d = src.reshape(e_rows, _ROW)
  dst2d = dst.reshape(e_rows, _ROW)
  ea16 = jnp.concatenate(
      [edge_attr, jnp.ones((e, 1), jnp.float32),
       jnp.zeros((e, 12), jnp.float32)], axis=1)
  if pad:
    ea16 = jnp.concatenate([ea16, jnp.zeros((pad, 16), jnp.float32)])
  ea16 = ea16.reshape(e_rows, _ROW, 16)

  z16 = jnp.zeros((n_acc, h_dim), jnp.float32)

  xl, xr, m1, c1 = pl.pallas_call(
      _pre_tc,
      out_shape=[
          jax.ShapeDtypeStruct((n, h_dim), jnp.float32),
          jax.ShapeDtypeStruct((n, h_dim), jnp.float32),
          jax.ShapeDtypeStruct((W_e1.shape[0], h_dim), jnp.float32),
          jax.ShapeDtypeStruct((1, h_dim), jnp.float32),
      ],
  )(x, W_l1, W_r1, W_e1, b_e1.reshape(1, d_in))

  seg1 = _seg_sum_sc(n_acc, e_rows, h_dim, with_ea=True)
  s1p, a4p = seg1(xl, src2d, dst2d, ea16, z16)

  h, hr = pl.pallas_call(
      _mid_tc,
      out_shape=[
          jax.ShapeDtypeStruct((n, h_dim), jnp.float32),
          jax.ShapeDtypeStruct((n, d_out), jnp.float32),
      ],
  )(s1p[0], s1p[1], a4p[0], a4p[1], xr, m1, c1,
    (b_l1 + b_r1).reshape(1, h_dim), W_r2)

  seg2 = _seg_sum_sc(n_acc, e_rows, h_dim, with_ea=False)
  (s2p,) = seg2(h, src2d, dst2d, z16)

  out = pl.pallas_call(
      _out_tc,
      out_shape=jax.ShapeDtypeStruct((n, d_out), jnp.float32),
  )(s2p[0], s2p[1], a4p[0], a4p[1], hr, W_e2,
    b_e2.reshape(1, h_dim), W_l2, (b_l2 + b_r2).reshape(1, d_out))
  return out


# speedup vs baseline: 7.2304x; 1.1573x over previous
import functools

import jax
import jax.numpy as jnp
from jax import lax
from jax.experimental import pallas as pl
from jax.experimental.pallas import tpu as pltpu
from jax.experimental.pallas import tpu_sc as plsc

_NC = 2
_NS = 16
_NW = _NC * _NS
_ROW = 128
_K = 8


def _seg_sum_sc(n_acc, e_rows, width, with_ea):
  rows_per_w = e_rows // _NW
  n_blocks = rows_per_w // _K
  zrows = n_acc // _NS

  mesh = plsc.VectorSubcoreMesh(
      core_axis_name="c", subcore_axis_name="s",
      num_cores=_NC, num_subcores=_NS)

  out_type = [jax.ShapeDtypeStruct((_NC, n_acc, width), jnp.float32)]
  scratch = [
      pltpu.VMEM((_K, _ROW), jnp.int32),
      pltpu.VMEM((_K, _ROW), jnp.int32),
      pltpu.VMEM((_K, _ROW, width), jnp.float32),
      pltpu.VMEM_SHARED((n_acc, width), jnp.float32),
      pltpu.SemaphoreType.DMA,
      pltpu.SemaphoreType.DMA,
  ]
  if with_ea:
    out_type.append(jax.ShapeDtypeStruct((_NC, n_acc, 16), jnp.float32))
    scratch.append(pltpu.VMEM((_K, _ROW, 16), jnp.float32))
    scratch.append(pltpu.VMEM_SHARED((n_acc, 16), jnp.float32))

  def body(*refs):
    if with_ea:
      (table, srcr, dstr, ear, z_s, s_out, a_out,
       src_v, dst_v, rows_v, acc_s, sem_g, sem_s, ea_v, acc_a) = refs
    else:
      (table, srcr, dstr, z_s, s_out,
       src_v, dst_v, rows_v, acc_s, sem_g, sem_s) = refs

    cid = lax.axis_index("c")
    sid = lax.axis_index("s")
    wid = sid * _NC + cid

    z0 = sid * zrows
    pltpu.sync_copy(z_s.at[pl.ds(z0, zrows)], acc_s.at[pl.ds(z0, zrows)])
    if with_ea:
      pltpu.sync_copy(z_s.at[pl.ds(z0, zrows)], acc_a.at[pl.ds(z0, zrows)])
    plsc.subcore_barrier()

    row0 = wid * rows_per_w

    @pl.loop(0, n_blocks)
    def _block(b):
      base = row0 + b * _K
      pltpu.sync_copy(srcr.at[pl.ds(base, _K)], src_v)
      pltpu.sync_copy(dstr.at[pl.ds(base, _K)], dst_v)
      if with_ea:
        pltpu.sync_copy(ear.at[pl.ds(base, _K)], ea_v)
      gets = [pltpu.async_copy(table.at[src_v.at[j]], rows_v.at[j], sem_g)
              for j in range(_K)]
      puts = []
      for j in range(_K):
        gets[j].wait()
        puts.append(pltpu.async_copy(rows_v.at[j], acc_s.at[dst_v.at[j]],
                                     sem_s, add=True))
        if with_ea:
          puts.append(pltpu.async_copy(ea_v.at[j], acc_a.at[dst_v.at[j]],
                                       sem_s, add=True))
      for d in puts:
        d.wait()

    plsc.subcore_barrier()

    pltpu.sync_copy(acc_s.at[pl.ds(z0, zrows)],
                    s_out.at[cid, pl.ds(z0, zrows)])
    if with_ea:
      pltpu.sync_copy(acc_a.at[pl.ds(z0, zrows)],
                      a_out.at[cid, pl.ds(z0, zrows)])

  return pl.kernel(
      body, out_type=out_type, mesh=mesh, scratch_types=scratch,
      compiler_params=pltpu.CompilerParams(use_tc_tiling_on_sc=False))


def _pre_tc(x_ref, wl_ref, wr_ref, we_ref, be_ref,
            xl_ref, xr_ref, m1_ref, c1_ref):
  xv = x_ref[...]
  wl = wl_ref[...]
  xl_ref[...] = jnp.dot(xv, wl, preferred_element_type=jnp.float32)
  xr_ref[...] = jnp.dot(xv, wr_ref[...], preferred_element_type=jnp.float32)
  m1_ref[...] = jnp.dot(we_ref[...], wl, preferred_element_type=jnp.float32)
  c1_ref[...] = jnp.dot(be_ref[...], wl, preferred_element_type=jnp.float32)


def _mid_tc(s1a_ref, s1b_ref, a4a_ref, a4b_ref, xr_ref, m1_ref, c1_ref,
            b1_ref, wr2_ref, h_ref, hr_ref):
  nn = h_ref.shape[0]
  a4 = a4a_ref[:nn, :] + a4b_ref[:nn, :]
  cnt = a4[:, 3:4]
  m1 = m1_ref[...]
  ce = (a4[:, 0:1] * m1[0:1, :] + a4[:, 1:2] * m1[1:2, :]
        + a4[:, 2:3] * m1[2:3, :])
  num = s1a_ref[:nn, :] + s1b_ref[:nn, :] + ce + cnt * c1_ref[...]
  h = jnp.maximum(num / jnp.maximum(cnt, 1.0) + xr_ref[...] + b1_ref[...], 0.0)
  h_ref[...] = h
  hr_ref[...] = jnp.dot(h, wr2_ref[...], preferred_element_type=jnp.float32)


def _out_tc(s2a_ref, s2b_ref, a4a_ref, a4b_ref, hr_ref, we2_ref, be2_ref,
            wl2_ref, b2_ref, o_ref):
  nn = o_ref.shape[0]
  a4 = a4a_ref[:nn, :] + a4b_ref[:nn, :]
  cnt = a4[:, 3:4]
  we2 = we2_ref[...]
  ce = (a4[:, 0:1] * we2[0:1, :] + a4[:, 1:2] * we2[1:2, :]
        + a4[:, 2:3] * we2[2:3, :])
  t = s2a_ref[:nn, :] + s2b_ref[:nn, :] + ce + cnt * be2_ref[...]
  agg = t / jnp.maximum(cnt, 1.0)
  o_ref[...] = (jnp.dot(agg, wl2_ref[...], preferred_element_type=jnp.float32)
                + hr_ref[...] + b2_ref[...])


def kernel(x, edge_index, edge_attr, W_e1, b_e1, W_l1, b_l1, W_r1, b_r1,
           W_e2, b_e2, W_l2, b_l2, W_r2, b_r2):
  n, d_in = x.shape
  e = edge_index.shape[1]
  h_dim = W_l1.shape[1]
  d_out = W_l2.shape[1]

  blk = _NW * _K * _ROW
  e_pad = -(-e // blk) * blk
  pad = e_pad - e
  e_rows = e_pad // _ROW
  n_acc = -(-(n + 1) // (_NS * 8)) * (_NS * 8)

  src = edge_index[0]
  dst = edge_index[1]
  if pad:
    src = jnp.concatenate([src, jnp.zeros((pad,), jnp.int32)])
    dst = jnp.concatenate([dst, jnp.full((pad,), n, jnp.int32)])
  src2d = src.reshape(e_rows, _ROW)
  dst2d = dst.reshape(e_rows, _ROW)
  ea16 = jnp.concatenate(
      [edge_attr, jnp.ones((e, 1), jnp.float32),
       jnp.zeros((e, 12), jnp.float32)], axis=1)
  if pad:
    ea16 = jnp.concatenate([ea16, jnp.zeros((pad, 16), jnp.float32)])
  ea16 = ea16.reshape(e_rows, _ROW, 16)

  z16 = jnp.zeros((n_acc, h_dim), jnp.float32)

  xl, xr, m1, c1 = pl.pallas_call(
      _pre_tc,
      out_shape=[
          jax.ShapeDtypeStruct((n, h_dim), jnp.float32),
          jax.ShapeDtypeStruct((n, h_dim), jnp.float32),
          jax.ShapeDtypeStruct((W_e1.shape[0], h_dim), jnp.float32),
          jax.ShapeDtypeStruct((1, h_dim), jnp.float32),
      ],
  )(x, W_l1, W_r1, W_e1, b_e1.reshape(1, d_in))

  seg1 = _seg_sum_sc(n_acc, e_rows, h_dim, with_ea=True)
  s1p, a4p = seg1(xl, src2d, dst2d, ea16, z16)

  h, hr = pl.pallas_call(
      _mid_tc,
      out_shape=[
          jax.ShapeDtypeStruct((n, h_dim), jnp.float32),
          jax.ShapeDtypeStruct((n, d_out), jnp.float32),
      ],
  )(s1p[0], s1p[1], a4p[0], a4p[1], xr, m1, c1,
    (b_l1 + b_r1).reshape(1, h_dim), W_r2)

  seg2 = _seg_sum_sc(n_acc, e_rows, h_dim, with_ea=False)
  (s2p,) = seg2(h, src2d, dst2d, z16)

  out = pl.pallas_call(
      _out_tc,
      out_shape=jax.ShapeDtypeStruct((n, d_out), jnp.float32),
  )(s2p[0], s2p[1], a4p[0], a4p[1], hr, W_e2,
    b_e2.reshape(1, h_dim), W_l2, (b_l2 + b_r2).reshape(1, d_out))
  return out


# speedup vs baseline: 7.6802x; 1.0622x over previous
import functools

import jax
import jax.numpy as jnp
from jax import lax
from jax.experimental import pallas as pl
from jax.experimental.pallas import tpu as pltpu
from jax.experimental.pallas import tpu_sc as plsc

_NC = 2
_NS = 16
_NW = _NC * _NS
_ROW = 1024


def _seg_sum_sc(n_acc, e_rows, width, with_ea):
  n_blocks = e_rows // _NW
  zrows = n_acc // _NS

  mesh = plsc.VectorSubcoreMesh(
      core_axis_name="c", subcore_axis_name="s",
      num_cores=_NC, num_subcores=_NS)

  out_type = [jax.ShapeDtypeStruct((_NC, n_acc, width), jnp.float32)]
  scratch = [
      pltpu.VMEM((n_blocks, _ROW), jnp.int32),
      pltpu.VMEM((n_blocks, _ROW), jnp.int32),
      pltpu.VMEM((2, _ROW, width), jnp.float32),
      pltpu.VMEM_SHARED((n_acc, width), jnp.float32),
      [pltpu.SemaphoreType.DMA] * 2,
      [pltpu.SemaphoreType.DMA] * 2,
  ]
  if with_ea:
    out_type.append(jax.ShapeDtypeStruct((_NC, n_acc, 16), jnp.float32))
    scratch.append(pltpu.VMEM((2, _ROW, 16), jnp.float32))
    scratch.append(pltpu.VMEM_SHARED((n_acc, 16), jnp.float32))
    scratch.append([pltpu.SemaphoreType.DMA] * 2)
    scratch.append([pltpu.SemaphoreType.DMA] * 2)

  def body(*refs):
    if with_ea:
      (table, srcr, dstr, ear, z_s, s_out, a_out,
       src_v, dst_v, rows_v, acc_s, sem_g, sem_s,
       ea_v, acc_a, sem_e, sem_a) = refs
    else:
      (table, srcr, dstr, z_s, s_out,
       src_v, dst_v, rows_v, acc_s, sem_g, sem_s) = refs

    cid = lax.axis_index("c")
    sid = lax.axis_index("s")
    wid = sid * _NC + cid

    z0 = sid * zrows
    pltpu.sync_copy(z_s.at[pl.ds(z0, zrows)], acc_s.at[pl.ds(z0, zrows)])
    if with_ea:
      pltpu.sync_copy(z_s.at[pl.ds(z0, zrows)], acc_a.at[pl.ds(z0, zrows)])
    plsc.subcore_barrier()

    row0 = wid * n_blocks
    pltpu.sync_copy(srcr.at[pl.ds(row0, n_blocks)], src_v)
    pltpu.sync_copy(dstr.at[pl.ds(row0, n_blocks)], dst_v)

    def fire_gather(b):
      return pltpu.async_copy(table.at[src_v.at[b]], rows_v.at[b % 2],
                              sem_g[b % 2])

    def fire_ea_load(b):
      return pltpu.async_copy(ear.at[row0 + b], ea_v.at[b % 2],
                              sem_e[b % 2])

    gd = [None] * n_blocks
    sd = [None] * n_blocks
    ed = [None] * n_blocks
    ad = [None] * n_blocks
    gd[0] = fire_gather(0)
    if with_ea:
      ed[0] = fire_ea_load(0)
    for b in range(n_blocks):
      if b + 1 < n_blocks:
        if b >= 1:
          sd[b - 1].wait()
          if with_ea:
            ad[b - 1].wait()
        gd[b + 1] = fire_gather(b + 1)
        if with_ea:
          ed[b + 1] = fire_ea_load(b + 1)
      gd[b].wait()
      sd[b] = pltpu.async_copy(rows_v.at[b % 2], acc_s.at[dst_v.at[b]],
                               sem_s[b % 2], add=True)
      if with_ea:
        ed[b].wait()
        ad[b] = pltpu.async_copy(ea_v.at[b % 2], acc_a.at[dst_v.at[b]],
                                 sem_a[b % 2], add=True)
    for b in range(max(n_blocks - 2, 0), n_blocks):
      sd[b].wait()
      if with_ea:
        ad[b].wait()

    plsc.subcore_barrier()

    pltpu.sync_copy(acc_s.at[pl.ds(z0, zrows)],
                    s_out.at[cid, pl.ds(z0, zrows)])
    if with_ea:
      pltpu.sync_copy(acc_a.at[pl.ds(z0, zrows)],
                      a_out.at[cid, pl.ds(z0, zrows)])

  return pl.kernel(
      body, out_type=out_type, mesh=mesh, scratch_types=scratch,
      compiler_params=pltpu.CompilerParams(use_tc_tiling_on_sc=False))


def _pre_tc(x_ref, wl_ref, wr_ref, we_ref, be_ref,
            xl_ref, xr_ref, m1_ref, c1_ref):
  xv = x_ref[...]
  wl = wl_ref[...]
  xl_ref[...] = jnp.dot(xv, wl, preferred_element_type=jnp.float32)
  xr_ref[...] = jnp.dot(xv, wr_ref[...], preferred_element_type=jnp.float32)
  m1_ref[...] = jnp.dot(we_ref[...], wl, preferred_element_type=jnp.float32)
  c1_ref[...] = jnp.dot(be_ref[...], wl, preferred_element_type=jnp.float32)


def _mid_tc(s1a_ref, s1b_ref, a4a_ref, a4b_ref, xr_ref, m1_ref, c1_ref,
            b1_ref, wr2_ref, h_ref, hr_ref):
  nn = h_ref.shape[0]
  a4 = a4a_ref[:nn, :] + a4b_ref[:nn, :]
  cnt = a4[:, 3:4]
  m1 = m1_ref[...]
  ce = (a4[:, 0:1] * m1[0:1, :] + a4[:, 1:2] * m1[1:2, :]
        + a4[:, 2:3] * m1[2:3, :])
  num = s1a_ref[:nn, :] + s1b_ref[:nn, :] + ce + cnt * c1_ref[...]
  h = jnp.maximum(num / jnp.maximum(cnt, 1.0) + xr_ref[...] + b1_ref[...], 0.0)
  h_ref[...] = h
  hr_ref[...] = jnp.dot(h, wr2_ref[...], preferred_element_type=jnp.float32)


def _out_tc(s2a_ref, s2b_ref, a4a_ref, a4b_ref, hr_ref, we2_ref, be2_ref,
            wl2_ref, b2_ref, o_ref):
  nn = o_ref.shape[0]
  a4 = a4a_ref[:nn, :] + a4b_ref[:nn, :]
  cnt = a4[:, 3:4]
  we2 = we2_ref[...]
  ce = (a4[:, 0:1] * we2[0:1, :] + a4[:, 1:2] * we2[1:2, :]
        + a4[:, 2:3] * we2[2:3, :])
  t = s2a_ref[:nn, :] + s2b_ref[:nn, :] + ce + cnt * be2_ref[...]
  agg = t / jnp.maximum(cnt, 1.0)
  o_ref[...] = (jnp.dot(agg, wl2_ref[...], preferred_element_type=jnp.float32)
                + hr_ref[...] + b2_ref[...])


def kernel(x, edge_index, edge_attr, W_e1, b_e1, W_l1, b_l1, W_r1, b_r1,
           W_e2, b_e2, W_l2, b_l2, W_r2, b_r2):
  n, d_in = x.shape
  e = edge_index.shape[1]
  h_dim = W_l1.shape[1]
  d_out = W_l2.shape[1]

  blk = _NW * _ROW
  e_pad = -(-e // blk) * blk
  pad = e_pad - e
  e_rows = e_pad // _ROW
  n_acc = -(-(n + 1) // (_NS * 8)) * (_NS * 8)

  src = edge_index[0]
  dst = edge_index[1]
  if pad:
    src = jnp.concatenate([src, jnp.zeros((pad,), jnp.int32)])
    dst = jnp.concatenate([dst, jnp.full((pad,), n, jnp.int32)])
  src2d = src.reshape(e_rows, _ROW)
  dst2d = dst.reshape(e_rows, _ROW)
  ea16 = jnp.concatenate(
      [edge_attr, jnp.ones((e, 1), jnp.float32),
       jnp.zeros((e, 12), jnp.float32)], axis=1)
  if pad:
    ea16 = jnp.concatenate([ea16, jnp.zeros((pad, 16), jnp.float32)])
  ea16 = ea16.reshape(e_rows, _ROW, 16)

  z16 = jnp.zeros((n_acc, h_dim), jnp.float32)

  xl, xr, m1, c1 = pl.pallas_call(
      _pre_tc,
      out_shape=[
          jax.ShapeDtypeStruct((n, h_dim), jnp.float32),
          jax.ShapeDtypeStruct((n, h_dim), jnp.float32),
          jax.ShapeDtypeStruct((W_e1.shape[0], h_dim), jnp.float32),
          jax.ShapeDtypeStruct((1, h_dim), jnp.float32),
      ],
  )(x, W_l1, W_r1, W_e1, b_e1.reshape(1, d_in))

  seg1 = _seg_sum_sc(n_acc, e_rows, h_dim, with_ea=True)
  s1p, a4p = seg1(xl, src2d, dst2d, ea16, z16)

  h, hr = pl.pallas_call(
      _mid_tc,
      out_shape=[
          jax.ShapeDtypeStruct((n, h_dim), jnp.float32),
          jax.ShapeDtypeStruct((n, d_out), jnp.float32),
      ],
  )(s1p[0], s1p[1], a4p[0], a4p[1], xr, m1, c1,
    (b_l1 + b_r1).reshape(1, h_dim), W_r2)

  seg2 = _seg_sum_sc(n_acc, e_rows, h_dim, with_ea=False)
  (s2p,) = seg2(h, src2d, dst2d, z16)

  out = pl.pallas_call(
      _out_tc,
      out_shape=jax.ShapeDtypeStruct((n, d_out), jnp.float32),
  )(s2p[0], s2p[1], a4p[0], a4p[1], hr, W_e2,
    b_e2.reshape(1, h_dim), W_l2, (b_l2 + b_r2).reshape(1, d_out))
  return out


# speedup vs baseline: 13.1308x; 1.7097x over previous
import functools

import jax
import jax.numpy as jnp
from jax import lax
from jax.experimental import pallas as pl
from jax.experimental.pallas import tpu as pltpu
from jax.experimental.pallas import tpu_sc as plsc

_NC = 2
_NS = 16
_NW = _NC * _NS
_ROW = 1024


def _seg_sum_sc(n_acc, n_blocks, width, with_ea):
  zrows = n_acc // _NS
  span = n_blocks * _ROW

  mesh = plsc.VectorSubcoreMesh(
      core_axis_name="c", subcore_axis_name="s",
      num_cores=_NC, num_subcores=_NS)

  out_type = [jax.ShapeDtypeStruct((_NC, n_acc, width), jnp.float32)]
  scratch = [
      pltpu.VMEM((span,), jnp.int32),
      pltpu.VMEM((span,), jnp.int32),
      pltpu.VMEM((2, _ROW, width), jnp.float32),
      pltpu.VMEM_SHARED((n_acc, width), jnp.float32),
      [pltpu.SemaphoreType.DMA] * 2,
      [pltpu.SemaphoreType.DMA] * 2,
  ]
  if with_ea:
    out_type.append(jax.ShapeDtypeStruct((_NC, n_acc, 16), jnp.float32))
    scratch.append(pltpu.VMEM((2, _ROW, 16), jnp.float32))
    scratch.append(pltpu.VMEM((3, 2, _ROW), jnp.float32))
    scratch.append(pltpu.VMEM_SHARED((n_acc, 16), jnp.float32))
    scratch.append([pltpu.SemaphoreType.DMA] * 2)
    scratch.append([pltpu.SemaphoreType.DMA] * 2)

  def body(*refs):
    if with_ea:
      (table, srcr, dstr, ea0r, ea1r, ea2r, z_s, s_out, a_out,
       src_v, dst_v, rows_v, acc_s, sem_g, sem_s,
       ea_v, eac_v, acc_a, sem_e, sem_a) = refs
      ear = (ea0r, ea1r, ea2r)
    else:
      (table, srcr, dstr, z_s, s_out,
       src_v, dst_v, rows_v, acc_s, sem_g, sem_s) = refs

    cid = lax.axis_index("c")
    sid = lax.axis_index("s")
    wid = sid * _NC + cid

    z0 = sid * zrows
    pltpu.sync_copy(z_s.at[pl.ds(z0, zrows)], acc_s.at[pl.ds(z0, zrows)])
    if with_ea:
      pltpu.sync_copy(z_s.at[pl.ds(z0, zrows)], acc_a.at[pl.ds(z0, zrows)])
      tmpl = jnp.where(lax.iota(jnp.int32, 16) == 3, 1.0, 0.0)
      for p in range(2):
        @pl.loop(0, _ROW)
        def _init(r, p=p):
          ea_v[p, r] = tmpl
    plsc.subcore_barrier()

    e0 = wid * span
    pltpu.sync_copy(srcr.at[pl.ds(e0, span)], src_v)
    pltpu.sync_copy(dstr.at[pl.ds(e0, span)], dst_v)

    def fire_gather(b):
      return pltpu.async_copy(table.at[src_v.at[pl.ds(b * _ROW, _ROW)]],
                              rows_v.at[b % 2], sem_g[b % 2])

    def fire_ea_loads(b):
      return [pltpu.async_copy(ear[c].at[pl.ds(e0 + b * _ROW, _ROW)],
                               eac_v.at[c, b % 2], sem_e[b % 2])
              for c in range(3)]

    def build_ea_rows(b):
      p = b % 2
      for g in range(_ROW // 16):
        ridx = lax.iota(jnp.int32, 16) + (g * 16)
        for c in range(3):
          vals = eac_v[c, p, pl.ds(g * 16, 16)]
          plsc.store_scatter(
              ea_v, [jnp.full((16,), p, jnp.int32), ridx,
                     jnp.full((16,), c, jnp.int32)], vals)

    gd = [None] * n_blocks
    sd = [None] * n_blocks
    ed = [None] * n_blocks
    ad = [None] * n_blocks
    gd[0] = fire_gather(0)
    if with_ea:
      ed[0] = fire_ea_loads(0)
    for b in range(n_blocks):
      if b + 1 < n_blocks:
        if b >= 1:
          sd[b - 1].wait()
          if with_ea:
            ad[b - 1].wait()
        gd[b + 1] = fire_gather(b + 1)
        if with_ea:
          ed[b + 1] = fire_ea_loads(b + 1)
      gd[b].wait()
      sd[b] = pltpu.async_copy(rows_v.at[b % 2],
                               acc_s.at[dst_v.at[pl.ds(b * _ROW, _ROW)]],
                               sem_s[b % 2], add=True)
      if with_ea:
        for d in ed[b]:
          d.wait()
        build_ea_rows(b)
        ad[b] = pltpu.async_copy(ea_v.at[b % 2],
                                 acc_a.at[dst_v.at[pl.ds(b * _ROW, _ROW)]],
                                 sem_a[b % 2], add=True)
    for b in range(max(n_blocks - 2, 0), n_blocks):
      sd[b].wait()
      if with_ea:
        ad[b].wait()

    plsc.subcore_barrier()

    pltpu.sync_copy(acc_s.at[pl.ds(z0, zrows)],
                    s_out.at[cid, pl.ds(z0, zrows)])
    if with_ea:
      pltpu.sync_copy(acc_a.at[pl.ds(z0, zrows)],
                      a_out.at[cid, pl.ds(z0, zrows)])

  return pl.kernel(
      body, out_type=out_type, mesh=mesh, scratch_types=scratch,
      compiler_params=pltpu.CompilerParams(use_tc_tiling_on_sc=False,
                                           needs_layout_passes=False))


def _pre_tc(x_ref, wl_ref, wr_ref, we_ref, be_ref,
            xl_ref, xr_ref, m1_ref, c1_ref):
  xv = x_ref[...]
  wl = wl_ref[...]
  xl_ref[...] = jnp.dot(xv, wl, preferred_element_type=jnp.float32)
  xr_ref[...] = jnp.dot(xv, wr_ref[...], preferred_element_type=jnp.float32)
  m1_ref[...] = jnp.dot(we_ref[...], wl, preferred_element_type=jnp.float32)
  c1_ref[...] = jnp.dot(be_ref[...], wl, preferred_element_type=jnp.float32)


def _mid_tc(s1a_ref, s1b_ref, a4a_ref, a4b_ref, xr_ref, m1_ref, c1_ref,
            b1_ref, wr2_ref, h_ref, hr_ref):
  nn = h_ref.shape[0]
  a4 = a4a_ref[:nn, :] + a4b_ref[:nn, :]
  cnt = a4[:, 3:4]
  m1 = m1_ref[...]
  ce = (a4[:, 0:1] * m1[0:1, :] + a4[:, 1:2] * m1[1:2, :]
        + a4[:, 2:3] * m1[2:3, :])
  num = s1a_ref[:nn, :] + s1b_ref[:nn, :] + ce + cnt * c1_ref[...]
  h = jnp.maximum(num / jnp.maximum(cnt, 1.0) + xr_ref[...] + b1_ref[...], 0.0)
  h_ref[...] = h
  hr_ref[...] = jnp.dot(h, wr2_ref[...], preferred_element_type=jnp.float32)


def _out_tc(s2a_ref, s2b_ref, a4a_ref, a4b_ref, hr_ref, we2_ref, be2_ref,
            wl2_ref, b2_ref, o_ref):
  nn = o_ref.shape[0]
  a4 = a4a_ref[:nn, :] + a4b_ref[:nn, :]
  cnt = a4[:, 3:4]
  we2 = we2_ref[...]
  ce = (a4[:, 0:1] * we2[0:1, :] + a4[:, 1:2] * we2[1:2, :]
        + a4[:, 2:3] * we2[2:3, :])
  t = s2a_ref[:nn, :] + s2b_ref[:nn, :] + ce + cnt * be2_ref[...]
  agg = t / jnp.maximum(cnt, 1.0)
  o_ref[...] = (jnp.dot(agg, wl2_ref[...], preferred_element_type=jnp.float32)
                + hr_ref[...] + b2_ref[...])


def kernel(x, edge_index, edge_attr, W_e1, b_e1, W_l1, b_l1, W_r1, b_r1,
           W_e2, b_e2, W_l2, b_l2, W_r2, b_r2):
  n, d_in = x.shape
  e = edge_index.shape[1]
  h_dim = W_l1.shape[1]
  d_out = W_l2.shape[1]

  blk = _NW * _ROW
  e_pad = -(-e // blk) * blk
  pad = e_pad - e
  n_blocks = e_pad // blk
  n_acc = -(-(n + 1) // (_NS * 8)) * (_NS * 8)

  src = edge_index[0]
  dst = edge_index[1]
  ea0 = edge_attr[:, 0]
  ea1 = edge_attr[:, 1]
  ea2 = edge_attr[:, 2]
  if pad:
    src = jnp.concatenate([src, jnp.zeros((pad,), jnp.int32)])
    dst = jnp.concatenate([dst, jnp.full((pad,), n, jnp.int32)])
    zpad = jnp.zeros((pad,), jnp.float32)
    ea0 = jnp.concatenate([ea0, zpad])
    ea1 = jnp.concatenate([ea1, zpad])
    ea2 = jnp.concatenate([ea2, zpad])

  z16 = jnp.zeros((n_acc, h_dim), jnp.float32)

  xl, xr, m1, c1 = pl.pallas_call(
      _pre_tc,
      out_shape=[
          jax.ShapeDtypeStruct((n, h_dim), jnp.float32),
          jax.ShapeDtypeStruct((n, h_dim), jnp.float32),
          jax.ShapeDtypeStruct((W_e1.shape[0], h_dim), jnp.float32),
          jax.ShapeDtypeStruct((1, h_dim), jnp.float32),
      ],
  )(x, W_l1, W_r1, W_e1, b_e1.reshape(1, d_in))

  seg1 = _seg_sum_sc(n_acc, n_blocks, h_dim, with_ea=True)
  s1p, a4p = seg1(xl, src, dst, ea0, ea1, ea2, z16)

  h, hr = pl.pallas_call(
      _mid_tc,
      out_shape=[
          jax.ShapeDtypeStruct((n, h_dim), jnp.float32),
          jax.ShapeDtypeStruct((n, d_out), jnp.float32),
      ],
  )(s1p[0], s1p[1], a4p[0], a4p[1], xr, m1, c1,
    (b_l1 + b_r1).reshape(1, h_dim), W_r2)

  seg2 = _seg_sum_sc(n_acc, n_blocks, h_dim, with_ea=False)
  (s2p,) = seg2(h, src, dst, z16)

  out = pl.pallas_call(
      _out_tc,
      out_shape=jax.ShapeDtypeStruct((n, d_out), jnp.float32),
  )(s2p[0], s2p[1], a4p[0], a4p[1], hr, W_e2,
    b_e2.reshape(1, h_dim), W_l2, (b_l2 + b_r2).reshape(1, d_out))
  return out


# speedup vs baseline: 13.6187x; 1.0372x over previous
import functools

import jax
import jax.numpy as jnp
from jax import lax
from jax.experimental import pallas as pl
from jax.experimental.pallas import tpu as pltpu
from jax.experimental.pallas import tpu_sc as plsc

_NC = 2
_NS = 16
_NW = _NC * _NS
_ROW = 1024


def _seg_sum_sc(n_acc, n_blocks, width, with_ea):
  zrows = n_acc // _NS
  span = n_blocks * _ROW

  mesh = plsc.VectorSubcoreMesh(
      core_axis_name="c", subcore_axis_name="s",
      num_cores=_NC, num_subcores=_NS)

  out_type = [jax.ShapeDtypeStruct((_NC, n_acc, width), jnp.float32)]
  scratch = [
      pltpu.VMEM((span,), jnp.int32),
      pltpu.VMEM((span,), jnp.int32),
      pltpu.VMEM((2, _ROW, width), jnp.float32),
      pltpu.VMEM_SHARED((n_acc, width), jnp.float32),
      [pltpu.SemaphoreType.DMA] * 2,
      [pltpu.SemaphoreType.DMA] * 2,
  ]
  if with_ea:
    out_type.append(jax.ShapeDtypeStruct((_NC, n_acc, 16), jnp.float32))
    scratch.append(pltpu.VMEM((2, _ROW, 16), jnp.float32))
    scratch.append(pltpu.VMEM((3, 2, _ROW), jnp.float32))
    scratch.append(pltpu.VMEM_SHARED((n_acc, 16), jnp.float32))
    scratch.append([pltpu.SemaphoreType.DMA] * 2)
    scratch.append([pltpu.SemaphoreType.DMA] * 2)

  def body(*refs):
    if with_ea:
      (table, srcr, dstr, ea0r, ea1r, ea2r, z_s, s_out, a_out,
       src_v, dst_v, rows_v, acc_s, sem_g, sem_s,
       ea_v, eac_v, acc_a, sem_e, sem_a) = refs
      ear = (ea0r, ea1r, ea2r)
    else:
      (table, srcr, dstr, z_s, s_out,
       src_v, dst_v, rows_v, acc_s, sem_g, sem_s) = refs

    cid = lax.axis_index("c")
    sid = lax.axis_index("s")
    wid = sid * _NC + cid

    z0 = sid * zrows
    pltpu.sync_copy(z_s.at[pl.ds(z0, zrows)], acc_s.at[pl.ds(z0, zrows)])
    if with_ea:
      pltpu.sync_copy(z_s.at[pl.ds(z0, zrows)], acc_a.at[pl.ds(z0, zrows)])
      tmpl = jnp.where(lax.iota(jnp.int32, 16) == 3, 1.0, 0.0)
      for p in range(2):
        @pl.loop(0, _ROW)
        def _init(r, p=p):
          ea_v[p, r] = tmpl
    plsc.subcore_barrier()

    e0 = wid * span
    pltpu.sync_copy(srcr.at[pl.ds(e0, span)], src_v)
    pltpu.sync_copy(dstr.at[pl.ds(e0, span)], dst_v)

    def fire_gather(b):
      return pltpu.async_copy(table.at[src_v.at[pl.ds(b * _ROW, _ROW)]],
                              rows_v.at[b % 2], sem_g[b % 2])

    def fire_ea_loads(b):
      return [pltpu.async_copy(ear[c].at[pl.ds(e0 + b * _ROW, _ROW)],
                               eac_v.at[c, b % 2], sem_e[b % 2])
              for c in range(3)]

    def build_ea_rows(b):
      p = b % 2
      pidx = jnp.full((16,), p, jnp.int32)
      iota = lax.iota(jnp.int32, 16)

      @pl.loop(0, _ROW // 16)
      def _grp(g):
        ridx = iota + g * 16
        for c in range(3):
          vals = eac_v[c, p, pl.ds(g * 16, 16)]
          plsc.store_scatter(
              ea_v, [pidx, ridx, jnp.full((16,), c, jnp.int32)], vals)

    gd = [None] * n_blocks
    sd = [None] * n_blocks
    ed = [None] * n_blocks
    ad = [None] * n_blocks
    gd[0] = fire_gather(0)
    if with_ea:
      ed[0] = fire_ea_loads(0)
    for b in range(n_blocks):
      if b + 1 < n_blocks:
        if b >= 1:
          sd[b - 1].wait()
          if with_ea:
            ad[b - 1].wait()
        gd[b + 1] = fire_gather(b + 1)
        if with_ea:
          ed[b + 1] = fire_ea_loads(b + 1)
      gd[b].wait()
      sd[b] = pltpu.async_copy(rows_v.at[b % 2],
                               acc_s.at[dst_v.at[pl.ds(b * _ROW, _ROW)]],
                               sem_s[b % 2], add=True)
      if with_ea:
        for d in ed[b]:
          d.wait()
        build_ea_rows(b)
        ad[b] = pltpu.async_copy(ea_v.at[b % 2],
                                 acc_a.at[dst_v.at[pl.ds(b * _ROW, _ROW)]],
                                 sem_a[b % 2], add=True)
    for b in range(max(n_blocks - 2, 0), n_blocks):
      sd[b].wait()
      if with_ea:
        ad[b].wait()

    plsc.subcore_barrier()

    pltpu.sync_copy(acc_s.at[pl.ds(z0, zrows)],
                    s_out.at[cid, pl.ds(z0, zrows)])
    if with_ea:
      pltpu.sync_copy(acc_a.at[pl.ds(z0, zrows)],
                      a_out.at[cid, pl.ds(z0, zrows)])

  return pl.kernel(
      body, out_type=out_type, mesh=mesh, scratch_types=scratch,
      compiler_params=pltpu.CompilerParams(use_tc_tiling_on_sc=False,
                                           needs_layout_passes=False))


def _pre_tc(x8_ref, wl8_ref, wr8_ref, wl_ref, we_ref, be_ref,
            xl_ref, xr_ref, m1_ref, c1_ref):
  xv = x8_ref[...]
  xl_ref[...] = jnp.dot(xv, wl8_ref[...], preferred_element_type=jnp.float32)
  xr_ref[...] = jnp.dot(xv, wr8_ref[...], preferred_element_type=jnp.float32)
  wl = wl_ref[...]
  m1_ref[...] = jnp.dot(we_ref[...], wl, preferred_element_type=jnp.float32)
  c1_ref[...] = jnp.dot(be_ref[...], wl, preferred_element_type=jnp.float32)


def _mid_tc(s1a_ref, s1b_ref, a4a_ref, a4b_ref, xr_ref, cb1_ref, bb_ref,
            b1_ref, h_ref):
  nr = h_ref.shape[0]
  a4 = a4a_ref[:nr, :] + a4b_ref[:nr, :]
  amat = jnp.dot(a4, cb1_ref[...], preferred_element_type=jnp.float32)
  cntb = jnp.dot(a4, bb_ref[...], preferred_element_type=jnp.float32)
  s1 = s1a_ref[:nr, :] + s1b_ref[:nr, :]
  num = s1 + amat
  h_ref[...] = jnp.maximum(
      num / jnp.maximum(cntb, 1.0) + xr_ref[...] + b1_ref[...], 0.0)


def _out_tc(s2a_ref, s2b_ref, a4a_ref, a4b_ref, h_ref, cb2_ref, bb_ref,
            wl2k_ref, wr2k_ref, b2_ref, o_ref):
  nr = o_ref.shape[0]
  a4 = a4a_ref[:nr, :] + a4b_ref[:nr, :]
  t = (s2a_ref[:nr, :] + s2b_ref[:nr, :]
       + jnp.dot(a4, cb2_ref[...], preferred_element_type=jnp.float32))
  cntb = jnp.dot(a4, bb_ref[...], preferred_element_type=jnp.float32)
  agg = t / jnp.maximum(cntb, 1.0)
  o_ref[...] = (jnp.dot(agg, wl2k_ref[...], preferred_element_type=jnp.float32)
                + jnp.dot(h_ref[...], wr2k_ref[...],
                          preferred_element_type=jnp.float32)
                + b2_ref[...])


def kernel(x, edge_index, edge_attr, W_e1, b_e1, W_l1, b_l1, W_r1, b_r1,
           W_e2, b_e2, W_l2, b_l2, W_r2, b_r2):
  n, d_in = x.shape
  e = edge_index.shape[1]
  h_dim = W_l1.shape[1]
  d_out = W_l2.shape[1]

  blk = _NW * _ROW
  e_pad = -(-e // blk) * blk
  pad = e_pad - e
  n_blocks = e_pad // blk
  n_acc = -(-(n + 1) // (_NS * 8)) * (_NS * 8)

  src = edge_index[0]
  dst = edge_index[1]
  ea0 = edge_attr[:, 0]
  ea1 = edge_attr[:, 1]
  ea2 = edge_attr[:, 2]
  if pad:
    src = jnp.concatenate([src, jnp.zeros((pad,), jnp.int32)])
    dst = jnp.concatenate([dst, jnp.full((pad,), n, jnp.int32)])
    zpad = jnp.zeros((pad,), jnp.float32)
    ea0 = jnp.concatenate([ea0, zpad])
    ea1 = jnp.concatenate([ea1, zpad])
    ea2 = jnp.concatenate([ea2, zpad])

  z16 = jnp.zeros((n_acc, h_dim), jnp.float32)

  eye8 = jnp.eye(8, dtype=jnp.float32)
  wl8 = jnp.kron(eye8, W_l1)
  wr8 = jnp.kron(eye8, W_r1)
  e3 = jnp.zeros((h_dim, h_dim), jnp.float32).at[3, :].set(1.0)
  bb = jnp.kron(eye8, e3)
  wl2k = jnp.kron(eye8, W_l2)
  wr2k = jnp.kron(eye8, W_r2)
  x8 = x.reshape(n // 8, 8 * d_in)
  nrow = n * h_dim // 128

  xl_p, xr_p, m1, c1 = pl.pallas_call(
      _pre_tc,
      out_shape=[
          jax.ShapeDtypeStruct((nrow, 128), jnp.float32),
          jax.ShapeDtypeStruct((nrow, 128), jnp.float32),
          jax.ShapeDtypeStruct((W_e1.shape[0], h_dim), jnp.float32),
          jax.ShapeDtypeStruct((1, h_dim), jnp.float32),
      ],
  )(x8, wl8, wr8, W_l1, W_e1, b_e1.reshape(1, d_in))
  xl = xl_p.reshape(n, h_dim)

  seg1 = _seg_sum_sc(n_acc, n_blocks, h_dim, with_ea=True)
  s1p, a4p = seg1(xl, src, dst, ea0, ea1, ea2, z16)
  prow = n_acc * h_dim // 128
  s1p = s1p.reshape(2, prow, 128)
  a4p = a4p.reshape(2, prow, 128)

  blk16 = jnp.concatenate(
      [m1, c1, jnp.zeros((h_dim - m1.shape[0] - 1, h_dim), jnp.float32)])
  cb1 = jnp.kron(eye8, blk16)
  blk16b = jnp.concatenate(
      [W_e2, b_e2.reshape(1, h_dim),
       jnp.zeros((h_dim - W_e2.shape[0] - 1, h_dim), jnp.float32)])
  cb2 = jnp.kron(eye8, blk16b)
  b1t = jnp.tile(b_l1 + b_r1, 8).reshape(1, 128)
  b2t = jnp.tile(b_l2 + b_r2, 8).reshape(1, 8 * d_out)

  h_p = pl.pallas_call(
      _mid_tc,
      out_shape=jax.ShapeDtypeStruct((nrow, 128), jnp.float32),
  )(s1p[0], s1p[1], a4p[0], a4p[1], xr_p, cb1, bb, b1t)
  h = h_p.reshape(n, h_dim)

  seg2 = _seg_sum_sc(n_acc, n_blocks, h_dim, with_ea=False)
  (s2p,) = seg2(h, src, dst, z16)
  s2p = s2p.reshape(2, prow, 128)

  out_p = pl.pallas_call(
      _out_tc,
      out_shape=jax.ShapeDtypeStruct((n // 8, 8 * d_out), jnp.float32),
  )(s2p[0], s2p[1], a4p[0], a4p[1], h_p, cb2, bb, wl2k, wr2k, b2t)
  return out_p.reshape(n, d_out)


# speedup vs baseline: 17.4012x; 1.2777x over previous
import functools

import jax
import jax.numpy as jnp
from jax import lax
from jax.experimental import pallas as pl
from jax.experimental.pallas import tpu as pltpu
from jax.experimental.pallas import tpu_sc as plsc

_NC = 2
_NS = 16
_NW = _NC * _NS
_ROW = 1024


def _seg_sum_sc(n_acc, n_blocks, width, with_ea):
  zrows = n_acc // _NS
  span = n_blocks * _ROW

  mesh = plsc.VectorSubcoreMesh(
      core_axis_name="c", subcore_axis_name="s",
      num_cores=_NC, num_subcores=_NS)

  out_type = [jax.ShapeDtypeStruct((_NC, n_acc, width), jnp.float32)]
  scratch = [
      pltpu.VMEM((span,), jnp.int32),
      pltpu.VMEM((span,), jnp.int32),
      pltpu.VMEM((2, _ROW, width), jnp.float32),
      pltpu.VMEM_SHARED((n_acc, width), jnp.float32),
      pltpu.VMEM_SHARED((n_acc, width), jnp.float32),
      [pltpu.SemaphoreType.DMA] * 2,
      [pltpu.SemaphoreType.DMA] * 2,
  ]
  if with_ea:
    out_type.append(jax.ShapeDtypeStruct((_NC, n_acc, 16), jnp.float32))
    scratch.append(pltpu.VMEM((2, _ROW, 16), jnp.float32))
    scratch.append(pltpu.VMEM((3, 2, _ROW), jnp.float32))
    scratch.append(pltpu.VMEM_SHARED((n_acc, 16), jnp.float32))
    scratch.append([pltpu.SemaphoreType.DMA] * 2)
    scratch.append([pltpu.SemaphoreType.DMA] * 2)

  def body(*refs):
    if with_ea:
      (table, srcr, dstr, ea0r, ea1r, ea2r, z_s, s_out, a_out,
       src_v, dst_v, rows_v, acc_s, tab_sp, sem_g, sem_s,
       ea_v, eac_v, acc_a, sem_e, sem_a) = refs
      ear = (ea0r, ea1r, ea2r)
    else:
      (table, srcr, dstr, z_s, s_out,
       src_v, dst_v, rows_v, acc_s, tab_sp, sem_g, sem_s) = refs

    cid = lax.axis_index("c")
    sid = lax.axis_index("s")
    wid = sid * _NC + cid

    z0 = sid * zrows
    pltpu.sync_copy(table.at[pl.ds(z0, zrows)], tab_sp.at[pl.ds(z0, zrows)])
    pltpu.sync_copy(z_s.at[pl.ds(z0, zrows)], acc_s.at[pl.ds(z0, zrows)])
    if with_ea:
      pltpu.sync_copy(z_s.at[pl.ds(z0, zrows)], acc_a.at[pl.ds(z0, zrows)])
      tmpl = jnp.where(lax.iota(jnp.int32, 16) == 3, 1.0, 0.0)
      for p in range(2):
        @pl.loop(0, _ROW)
        def _init(r, p=p):
          ea_v[p, r] = tmpl
    plsc.subcore_barrier()

    e0 = wid * span
    pltpu.sync_copy(srcr.at[pl.ds(e0, span)], src_v)
    pltpu.sync_copy(dstr.at[pl.ds(e0, span)], dst_v)

    def fire_gather(b):
      return pltpu.async_copy(tab_sp.at[src_v.at[pl.ds(b * _ROW, _ROW)]],
                              rows_v.at[b % 2], sem_g[b % 2])

    def fire_ea_loads(b):
      return [pltpu.async_copy(ear[c].at[pl.ds(e0 + b * _ROW, _ROW)],
                               eac_v.at[c, b % 2], sem_e[b % 2])
              for c in range(3)]

    def build_ea_rows(b):
      p = b % 2
      pidx = jnp.full((16,), p, jnp.int32)
      iota = lax.iota(jnp.int32, 16)

      @pl.loop(0, _ROW // 16)
      def _grp(g):
        ridx = iota + g * 16
        for c in range(3):
          vals = eac_v[c, p, pl.ds(g * 16, 16)]
          plsc.store_scatter(
              ea_v, [pidx, ridx, jnp.full((16,), c, jnp.int32)], vals)

    gd = [None] * n_blocks
    sd = [None] * n_blocks
    ed = [None] * n_blocks
    ad = [None] * n_blocks
    gd[0] = fire_gather(0)
    if with_ea:
      ed[0] = fire_ea_loads(0)
    for b in range(n_blocks):
      if b + 1 < n_blocks:
        if b >= 1:
          sd[b - 1].wait()
          if with_ea:
            ad[b - 1].wait()
        gd[b + 1] = fire_gather(b + 1)
        if with_ea:
          ed[b + 1] = fire_ea_loads(b + 1)
      gd[b].wait()
      sd[b] = pltpu.async_copy(rows_v.at[b % 2],
                               acc_s.at[dst_v.at[pl.ds(b * _ROW, _ROW)]],
                               sem_s[b % 2], add=True)
      if with_ea:
        for d in ed[b]:
          d.wait()
        build_ea_rows(b)
        ad[b] = pltpu.async_copy(ea_v.at[b % 2],
                                 acc_a.at[dst_v.at[pl.ds(b * _ROW, _ROW)]],
                                 sem_a[b % 2], add=True)
    for b in range(max(n_blocks - 2, 0), n_blocks):
      sd[b].wait()
      if with_ea:
        ad[b].wait()

    plsc.subcore_barrier()

    pltpu.sync_copy(acc_s.at[pl.ds(z0, zrows)],
                    s_out.at[cid, pl.ds(z0, zrows)])
    if with_ea:
      pltpu.sync_copy(acc_a.at[pl.ds(z0, zrows)],
                      a_out.at[cid, pl.ds(z0, zrows)])

  return pl.kernel(
      body, out_type=out_type, mesh=mesh, scratch_types=scratch,
      compiler_params=pltpu.CompilerParams(use_tc_tiling_on_sc=False,
                                           needs_layout_passes=False))


def _pre_tc(x8_ref, wl8_ref, wr8_ref, wl_ref, we_ref, be_ref,
            xl_ref, xr_ref, m1_ref, c1_ref):
  xv = x8_ref[...]
  xl_ref[...] = jnp.dot(xv, wl8_ref[...], preferred_element_type=jnp.float32)
  xr_ref[...] = jnp.dot(xv, wr8_ref[...], preferred_element_type=jnp.float32)
  wl = wl_ref[...]
  m1_ref[...] = jnp.dot(we_ref[...], wl, preferred_element_type=jnp.float32)
  c1_ref[...] = jnp.dot(be_ref[...], wl, preferred_element_type=jnp.float32)


def _mid_tc(s1a_ref, s1b_ref, a4a_ref, a4b_ref, xr_ref, cb1_ref, bb_ref,
            b1_ref, h_ref):
  nr = h_ref.shape[0]
  a4 = a4a_ref[:nr, :] + a4b_ref[:nr, :]
  amat = jnp.dot(a4, cb1_ref[...], preferred_element_type=jnp.float32)
  cntb = jnp.dot(a4, bb_ref[...], preferred_element_type=jnp.float32)
  s1 = s1a_ref[:nr, :] + s1b_ref[:nr, :]
  num = s1 + amat
  h_ref[...] = jnp.maximum(
      num / jnp.maximum(cntb, 1.0) + xr_ref[...] + b1_ref[...], 0.0)


def _out_tc(s2a_ref, s2b_ref, a4a_ref, a4b_ref, h_ref, cb2_ref, bb_ref,
            wl2k_ref, wr2k_ref, b2_ref, o_ref):
  nr = o_ref.shape[0]
  a4 = a4a_ref[:nr, :] + a4b_ref[:nr, :]
  t = (s2a_ref[:nr, :] + s2b_ref[:nr, :]
       + jnp.dot(a4, cb2_ref[...], preferred_element_type=jnp.float32))
  cntb = jnp.dot(a4, bb_ref[...], preferred_element_type=jnp.float32)
  agg = t / jnp.maximum(cntb, 1.0)
  o_ref[...] = (jnp.dot(agg, wl2k_ref[...], preferred_element_type=jnp.float32)
                + jnp.dot(h_ref[...], wr2k_ref[...],
                          preferred_element_type=jnp.float32)
                + b2_ref[...])


def kernel(x, edge_index, edge_attr, W_e1, b_e1, W_l1, b_l1, W_r1, b_r1,
           W_e2, b_e2, W_l2, b_l2, W_r2, b_r2):
  n, d_in = x.shape
  e = edge_index.shape[1]
  h_dim = W_l1.shape[1]
  d_out = W_l2.shape[1]

  blk = _NW * _ROW
  e_pad = -(-e // blk) * blk
  pad = e_pad - e
  n_blocks = e_pad // blk
  n_acc = -(-(n + 1) // (_NS * 8)) * (_NS * 8)

  src = edge_index[0]
  dst = edge_index[1]
  ea0 = edge_attr[:, 0]
  ea1 = edge_attr[:, 1]
  ea2 = edge_attr[:, 2]
  if pad:
    src = jnp.concatenate([src, jnp.zeros((pad,), jnp.int32)])
    dst = jnp.concatenate([dst, jnp.full((pad,), n, jnp.int32)])
    zpad = jnp.zeros((pad,), jnp.float32)
    ea0 = jnp.concatenate([ea0, zpad])
    ea1 = jnp.concatenate([ea1, zpad])
    ea2 = jnp.concatenate([ea2, zpad])

  z16 = jnp.zeros((n_acc, h_dim), jnp.float32)

  eye8 = jnp.eye(8, dtype=jnp.float32)
  wl8 = jnp.kron(eye8, W_l1)
  wr8 = jnp.kron(eye8, W_r1)
  e3 = jnp.zeros((h_dim, h_dim), jnp.float32).at[3, :].set(1.0)
  bb = jnp.kron(eye8, e3)
  wl2k = jnp.kron(eye8, W_l2)
  wr2k = jnp.kron(eye8, W_r2)
  x8 = x.reshape(n // 8, 8 * d_in)
  nrow = n * h_dim // 128

  xl_p, xr_p, m1, c1 = pl.pallas_call(
      _pre_tc,
      out_shape=[
          jax.ShapeDtypeStruct((nrow, 128), jnp.float32),
          jax.ShapeDtypeStruct((nrow, 128), jnp.float32),
          jax.ShapeDtypeStruct((W_e1.shape[0], h_dim), jnp.float32),
          jax.ShapeDtypeStruct((1, h_dim), jnp.float32),
      ],
  )(x8, wl8, wr8, W_l1, W_e1, b_e1.reshape(1, d_in))
  prow = n_acc * h_dim // 128
  padrows = prow - n * h_dim // 128
  xl = jnp.concatenate(
      [xl_p, jnp.zeros((padrows, 128), jnp.float32)]).reshape(n_acc, h_dim)

  seg1 = _seg_sum_sc(n_acc, n_blocks, h_dim, with_ea=True)
  s1p, a4p = seg1(xl, src, dst, ea0, ea1, ea2, z16)
  s1p = s1p.reshape(2, prow, 128)
  a4p = a4p.reshape(2, prow, 128)

  blk16 = jnp.concatenate(
      [m1, c1, jnp.zeros((h_dim - m1.shape[0] - 1, h_dim), jnp.float32)])
  cb1 = jnp.kron(eye8, blk16)
  blk16b = jnp.concatenate(
      [W_e2, b_e2.reshape(1, h_dim),
       jnp.zeros((h_dim - W_e2.shape[0] - 1, h_dim), jnp.float32)])
  cb2 = jnp.kron(eye8, blk16b)
  b1t = jnp.tile(b_l1 + b_r1, 8).reshape(1, 128)
  b2t = jnp.tile(b_l2 + b_r2, 8).reshape(1, 8 * d_out)

  h_p = pl.pallas_call(
      _mid_tc,
      out_shape=jax.ShapeDtypeStruct((nrow, 128), jnp.float32),
  )(s1p[0], s1p[1], a4p[0], a4p[1], xr_p, cb1, bb, b1t)
  h = jnp.concatenate(
      [h_p, jnp.zeros((padrows, 128), jnp.float32)]).reshape(n_acc, h_dim)

  seg2 = _seg_sum_sc(n_acc, n_blocks, h_dim, with_ea=False)
  (s2p,) = seg2(h, src, dst, z16)
  s2p = s2p.reshape(2, prow, 128)

  out_p = pl.pallas_call(
      _out_tc,
      out_shape=jax.ShapeDtypeStruct((n // 8, 8 * d_out), jnp.float32),
  )(s2p[0], s2p[1], a4p[0], a4p[1], h_p, cb2, bb, wl2k, wr2k, b2t)
  return out_p.reshape(n, d_out)
